# baseline (speedup 1.0000x reference)
import functools

import jax
import jax.numpy as jnp
from jax import lax
from jax.experimental import pallas as pl
from jax.experimental.pallas import tpu as pltpu
from jax.experimental.pallas import tpu_sc as plsc

BATCH = 4096
SEQ = 200
EMB = 64

NC, NS = 2, 16
NW = NC * NS
BPW = BATCH // NW
NBUF = 4


def _emb_body(idx_hbm, tab_hbm, out_hbm, idx_v, src_v, *sems):
    w = lax.axis_index("s") * NC + lax.axis_index("c")
    sem_g = sems[:NBUF]
    sem_o = sems[NBUF:]

    pltpu.sync_copy(idx_hbm.at[w], idx_v)

    def fire_gather(b, s):
        pltpu.async_copy(tab_hbm.at[idx_v.at[s]], src_v.at[b], sem_g[b])

    def drain_gather(b):
        pltpu.make_async_copy(out_hbm.at[0, w], src_v.at[b], sem_g[b]).wait()

    def fire_out(b, s):
        pltpu.async_copy(src_v.at[b], out_hbm.at[s, w], sem_o[b])

    def drain_out(b):
        pltpu.make_async_copy(out_hbm.at[0, w], src_v.at[b], sem_o[b]).wait()

    fire_gather(0, 0)

    @pl.loop(0, SEQ, step=NBUF)
    def _(g):
        for b in range(NBUF):
            s = g + b
            bn = (b + 1) % NBUF

            @pl.when(s + 1 < SEQ)
            def _():
                fire_gather(bn, s + 1)

            drain_gather(b)

            @pl.when(s >= NBUF)
            def _():
                drain_out(b)

            fire_out(b, s)

    for b in range(NBUF):
        drain_out(b)


@jax.jit
def kernel(input_tensor, weight):
    idx = (
        input_tensor.reshape(NW, BPW, SEQ).transpose(0, 2, 1).astype(jnp.int32)
    )
    mesh = plsc.VectorSubcoreMesh(
        core_axis_name="c", subcore_axis_name="s", num_cores=NC, num_subcores=NS
    )
    out4 = pl.kernel(
        _emb_body,
        out_type=jax.ShapeDtypeStruct((SEQ, NW, BPW, EMB), jnp.float32),
        mesh=mesh,
        scratch_types=[
            pltpu.VMEM((SEQ, BPW), jnp.int32),
            pltpu.VMEM((NBUF, BPW, EMB), jnp.float32),
        ]
        + [pltpu.SemaphoreType.DMA] * (2 * NBUF),
        compiler_params=pltpu.CompilerParams(
            use_tc_tiling_on_sc=False, needs_layout_passes=False
        ),
    )(idx, weight)
    return out4.transpose(1, 2, 0, 3).reshape(BATCH, SEQ, EMB)
